# baseline (speedup 1.0000x reference)
import functools

import jax
import jax.numpy as jnp
from jax import lax
from jax.experimental import pallas as pl
from jax.experimental.pallas import tpu as pltpu
from jax.experimental.pallas import tpu_sc as plsc


DP = 16


def _project_table(table, wpt):
    n, d = table.shape
    bm = 2000
    assert n % bm == 0

    def mm(xb, rb, ob):
        ob[...] = jnp.dot(xb[...], rb[...],
                          preferred_element_type=jnp.float32,
                          precision=jax.lax.Precision.HIGHEST)

    return pl.pallas_call(
        mm,
        grid=(n // bm,),
        in_specs=[
            pl.BlockSpec((bm, d), lambda i: (i, 0)),
            pl.BlockSpec((d, DP), lambda i: (0, 0)),
        ],
        out_specs=pl.BlockSpec((bm, DP), lambda i: (i, 0)),
        out_shape=jax.ShapeDtypeStruct((n, DP), jnp.float32),
    )(table, wpt)


def _sc_pool(xf, slf, p_tab, bp, b, l):
    info = plsc.get_sparse_core_info()
    nc, ns = info.num_cores, info.num_subcores
    nw = nc * ns
    rows_per_w = b // nw
    gb = 16
    chunk = gb * l
    nch = rows_per_w // gb
    ng = chunk // 128

    mesh = plsc.VectorSubcoreMesh(core_axis_name="c", subcore_axis_name="s")

    @functools.partial(
        pl.kernel,
        mesh=mesh,
        out_type=jax.ShapeDtypeStruct((b, DP), jnp.float32),
        compiler_params=pltpu.CompilerParams(use_tc_tiling_on_sc=False),
        scratch_types=[
            pltpu.VMEM((ng, 128), jnp.int32),
            pltpu.VMEM((ng, 128), jnp.int32),
            pltpu.VMEM((chunk, DP), jnp.float32),
            pltpu.VMEM((chunk, DP), jnp.float32),
            pltpu.VMEM((rows_per_w,), jnp.float32),
            pltpu.VMEM((rows_per_w, DP), jnp.float32),
            pltpu.VMEM((DP,), jnp.float32),
            pltpu.SemaphoreType.DMA,
            pltpu.SemaphoreType.DMA,
        ],
    )
    def body(xf_hbm, sl_hbm, p_hbm, bp_hbm, out_hbm,
             idx0, idx1, rows0, rows1, sl_v, out_v, bp_v, sem0, sem1):
        wid = lax.axis_index("s") * nc + lax.axis_index("c")
        b0 = wid * rows_per_w
        xrow0 = (b0 * l) // 128
        pltpu.sync_copy(sl_hbm.at[pl.ds(b0, rows_per_w)], sl_v)
        pltpu.sync_copy(bp_hbm, bp_v)
        bias = bp_v[...]

        def fire(g, idxb, rowsb, semb):
            pltpu.sync_copy(xf_hbm.at[pl.ds(xrow0 + g * ng, ng)], idxb)
            for j in range(ng):
                pltpu.async_copy(p_hbm.at[idxb.at[j]],
                                 rowsb.at[pl.ds(j * 128, 128)], semb)

        def drain(rowsb, semb):
            pltpu.make_async_copy(p_hbm.at[pl.ds(0, chunk)], rowsb,
                                  semb).wait()

        def accum(g, rowsb):
            sv = sl_v[pl.ds(g * gb, 16)]
            for r in range(gb):
                def lbody(j, accs, r=r):
                    base = r * l + j * 8
                    return tuple(accs[k] + rowsb[base + k] for k in range(8))
                z = jnp.zeros((DP,), jnp.float32)
                accs = lax.fori_loop(0, l // 8, lbody, (z,) * 8)
                acc = (((accs[0] + accs[1]) + (accs[2] + accs[3]))
                       + ((accs[4] + accs[5]) + (accs[6] + accs[7])))
                out_v[g * gb + r] = acc / sv[r] + bias

        fire(0, idx0, rows0, sem0)

        def pair_body(i, carry):
            g0 = i * 2
            fire(g0 + 1, idx1, rows1, sem1)
            drain(rows0, sem0)
            accum(g0, rows0)

            @pl.when(g0 + 2 < nch)
            def _():
                fire(g0 + 2, idx0, rows0, sem0)

            drain(rows1, sem1)
            accum(g0 + 1, rows1)
            return carry

        lax.fori_loop(0, nch // 2, pair_body, 0)
        pltpu.sync_copy(out_v, out_hbm.at[pl.ds(b0, rows_per_w)])

    return body(xf, slf, p_tab, bp)


def kernel(x, sl, table, W, b):
    bsz, l = x.shape
    n_cls = W.shape[0]
    wp = jnp.zeros((DP, table.shape[1]), jnp.float32).at[:n_cls].set(W)
    p_tab = _project_table(table, wp.T)
    bp = jnp.zeros((DP,), jnp.float32).at[:n_cls].set(b)
    s = _sc_pool(x.reshape(bsz * l // 128, 128), sl.astype(jnp.float32),
                 p_tab, bp, bsz, l)
    return s[:, :n_cls]


# speedup vs baseline: 2.0212x; 2.0212x over previous
import functools

import jax
import jax.numpy as jnp
from jax import lax
from jax.experimental import pallas as pl
from jax.experimental.pallas import tpu as pltpu
from jax.experimental.pallas import tpu_sc as plsc


DP = 16


def _sc_pool_classify(xf, slf, table, wrows, bp, b, l):
    n, d = table.shape
    info = plsc.get_sparse_core_info()
    nc, ns = info.num_cores, info.num_subcores
    nw = nc * ns
    rows_per_w = b // nw
    gb = 8
    chunk = gb * l
    nch = rows_per_w // gb

    mesh = plsc.VectorSubcoreMesh(core_axis_name="c", subcore_axis_name="s")

    @functools.partial(
        pl.kernel,
        mesh=mesh,
        out_type=jax.ShapeDtypeStruct((b, DP), jnp.float32),
        compiler_params=pltpu.CompilerParams(use_tc_tiling_on_sc=False),
        scratch_types=[
            pltpu.VMEM((chunk,), jnp.int32),
            pltpu.VMEM((chunk,), jnp.int32),
            pltpu.VMEM((chunk, 32), jnp.float32),
            pltpu.VMEM((chunk, 32), jnp.float32),
            pltpu.VMEM((rows_per_w,), jnp.float32),
            pltpu.VMEM((rows_per_w, DP), jnp.float32),
            pltpu.VMEM((4, DP), jnp.float32),
            pltpu.VMEM((DP,), jnp.float32),
            pltpu.SemaphoreType.DMA,
            pltpu.SemaphoreType.DMA,
        ],
    )
    def body(xf_hbm, sl_hbm, tab_hbm, w_hbm, bp_hbm, out_hbm,
             idx0, idx1, rows0, rows1, sl_v, out_v, w_v, bp_v, sem0, sem1):
        wid = lax.axis_index("s") * nc + lax.axis_index("c")
        b0 = wid * rows_per_w
        i0 = b0 * l
        pltpu.sync_copy(sl_hbm.at[pl.ds(b0, rows_per_w)], sl_v)
        pltpu.sync_copy(w_hbm, w_v)
        pltpu.sync_copy(bp_hbm, bp_v)
        w0a = w_v[0]
        w0b = w_v[1]
        w1a = w_v[2]
        w1b = w_v[3]
        bias = bp_v[...]
        lane = lax.iota(jnp.int32, 16)

        def allsum(v):
            for sh in (8, 4, 2, 1):
                v = v + v.at[jnp.bitwise_xor(lane, sh)].get(
                    mode="promise_in_bounds")
            return v

        def fire(g, idxb, rowsb, semb):
            pltpu.sync_copy(xf_hbm.at[pl.ds(i0 + g * chunk, chunk)], idxb)
            pltpu.async_copy(tab_hbm.at[idxb], rowsb, semb)

        def drain(rowsb, semb):
            pltpu.make_async_copy(tab_hbm.at[pl.ds(0, chunk)], rowsb,
                                  semb).wait()

        def accum(g, rowsb):
            sv = sl_v[pl.ds(g * gb, 16)]
            for r in range(gb):
                def lbody(j, accs, r=r):
                    base = r * l + j * 4
                    new = []
                    for k in range(4):
                        new.append(accs[k] + rowsb[base + k, pl.ds(0, 16)])
                    for k in range(4):
                        new.append(accs[4 + k] + rowsb[base + k,
                                                       pl.ds(16, 16)])
                    return tuple(new)
                z = jnp.zeros((DP,), jnp.float32)
                accs = lax.fori_loop(0, l // 4, lbody, (z,) * 8)
                acc_a = (accs[0] + accs[1]) + (accs[2] + accs[3])
                acc_b = (accs[4] + accs[5]) + (accs[6] + accs[7])
                s0 = allsum(acc_a * w0a + acc_b * w0b)
                s1 = allsum(acc_a * w1a + acc_b * w1b)
                res = jnp.where(lane == 0, s0,
                                jnp.where(lane == 1, s1, 0.0))
                out_v[g * gb + r] = res / sv[r] + bias

        fire(0, idx0, rows0, sem0)

        def pair_body(i, carry):
            g0 = i * 2
            fire(g0 + 1, idx1, rows1, sem1)
            drain(rows0, sem0)
            accum(g0, rows0)

            @pl.when(g0 + 2 < nch)
            def _():
                fire(g0 + 2, idx0, rows0, sem0)

            drain(rows1, sem1)
            accum(g0 + 1, rows1)
            return carry

        lax.fori_loop(0, nch // 2, pair_body, 0)
        pltpu.sync_copy(out_v, out_hbm.at[pl.ds(b0, rows_per_w)])

    return body(xf, slf, table, wrows, bp)


def kernel(x, sl, table, W, b):
    bsz, l = x.shape
    n_cls = W.shape[0]
    d = table.shape[1]
    wrows = W.reshape(n_cls * d // DP, DP)
    bp = jnp.zeros((DP,), jnp.float32).at[:n_cls].set(b)
    s = _sc_pool_classify(x.reshape(bsz * l), sl.astype(jnp.float32),
                          table, wrows, bp, bsz, l)
    return s[:, :n_cls]
